# baseline (speedup 1.0000x reference)
import functools

import jax
import jax.numpy as jnp
from jax import lax
from jax.experimental import pallas as pl
from jax.experimental.pallas import tpu as pltpu
from jax.experimental.pallas import tpu_sc as plsc

B, V, L, D, R = 4096, 100000, 50, 64, 5
LP = 56

NC, NS = 2, 16
NW = NC * NS
BPW = B // NW
NCH = D // 16

VB = 2000


def _prep_body(ft_ref, wuv_ref, ra_ref, wa_ref, fp_ref, rp_ref):
    fp_ref[...] = jnp.dot(ft_ref[...], wuv_ref[...],
                          preferred_element_type=jnp.float32)
    rp_ref[...] = jnp.dot(ra_ref[...], wa_ref[...],
                          preferred_element_type=jnp.float32)


_prep_call = pl.pallas_call(
    _prep_body,
    grid=(V // VB,),
    in_specs=[
        pl.BlockSpec((VB, D), lambda i: (i, 0)),
        pl.BlockSpec((D, D), lambda i: (0, 0)),
        pl.BlockSpec((8, D + 1), lambda i: (0, 0)),
        pl.BlockSpec((D + 1, D), lambda i: (0, 0)),
    ],
    out_specs=[
        pl.BlockSpec((VB, D), lambda i: (i, 0)),
        pl.BlockSpec((8, D), lambda i: (0, 0)),
    ],
    out_shape=[
        jax.ShapeDtypeStruct((V, D), jnp.float32),
        jax.ShapeDtypeStruct((8, D), jnp.float32),
    ],
)

_sc_mesh = plsc.VectorSubcoreMesh(core_axis_name="c", subcore_axis_name="s")


@functools.partial(
    pl.kernel,
    mesh=_sc_mesh,
    compiler_params=pltpu.CompilerParams(use_tc_tiling_on_sc=False),
    out_type=[
        jax.ShapeDtypeStruct((B, D), jnp.float32),
        jax.ShapeDtypeStruct((B, D), jnp.float32),
    ],
    scratch_types=[
        pltpu.VMEM((BPW,), jnp.int32),
        pltpu.VMEM((BPW, LP), jnp.int32),
        pltpu.VMEM((BPW, LP), jnp.int32),
        pltpu.VMEM((8, LP, D), jnp.float32),
        pltpu.VMEM((8, LP, D), jnp.float32),
        pltpu.VMEM((BPW, D), jnp.float32),
        pltpu.VMEM((BPW, D), jnp.float32),
        pltpu.SemaphoreType.DMA,
        pltpu.SemaphoreType.DMA,
        pltpu.SemaphoreType.DMA,
    ],
)
def _sc_agg(nodes_hbm, hu_hbm, hr_hbm, fproj_hbm, rproj_hbm, ftab_hbm,
            neigh_hbm, self_hbm,
            idx_v, hu_v, hr_v, fring, rring, acc_v, sbuf,
            semf, semr, sem2):
    NBUF = 8
    wid = lax.axis_index("s") * NC + lax.axis_index("c")
    base = wid * BPW

    pltpu.sync_copy(nodes_hbm.at[pl.ds(base, BPW)], idx_v)
    self_dma = pltpu.async_copy(ftab_hbm.at[idx_v], sbuf, sem2)
    pltpu.async_copy(hu_hbm.at[idx_v], hu_v, semf).wait()
    pltpu.async_copy(hr_hbm.at[idx_v], hr_v, semr).wait()

    zero = jnp.zeros((16,), jnp.float32)

    def _fire(i, b):
        pltpu.async_copy(fproj_hbm.at[hu_v.at[i]], fring.at[b], semf)
        pltpu.async_copy(rproj_hbm.at[hr_v.at[i]], rring.at[b], semr)

    def _drain(b):
        pltpu.make_async_copy(fproj_hbm.at[hu_v.at[0]], fring.at[b],
                              semf).wait()
        pltpu.make_async_copy(rproj_hbm.at[hr_v.at[0]], rring.at[b],
                              semr).wait()

    for b in range(NBUF):
        _fire(b, b)

    def _group(g, _):
        i0 = g * NBUF
        for b in range(NBUF):
            i = i0 + b
            _drain(b)
            fbuf = fring.at[b]
            rbuf = rring.at[b]

            def _lstep(l, acc):
                out = []
                for k in range(NCH):
                    s = pl.ds(16 * k, 16)
                    out.append(acc[k] +
                               jnp.maximum(fbuf[l, s] + rbuf[l, s], 0.0))
                return tuple(out)

            acc = lax.fori_loop(0, L, _lstep, (zero,) * NCH)
            for k in range(NCH):
                acc_v[i, pl.ds(16 * k, 16)] = acc[k]
            pltpu.async_copy(
                fproj_hbm.at[hu_v.at[jnp.minimum(i + NBUF, BPW - 1)]],
                fring.at[b], semf)
            pltpu.async_copy(
                rproj_hbm.at[hr_v.at[jnp.minimum(i + NBUF, BPW - 1)]],
                rring.at[b], semr)
        return 0

    lax.fori_loop(0, BPW // NBUF, _group, 0)

    for b in range(NBUF):
        _drain(b)

    pltpu.sync_copy(acc_v, neigh_hbm.at[pl.ds(base, BPW)])
    self_dma.wait()
    pltpu.sync_copy(sbuf, self_hbm.at[pl.ds(base, BPW)])


BN = 512


def _final_body(s_ref, n_ref, ws_ref, wn_ref, b_ref, o_ref):
    acc = jnp.dot(s_ref[...], ws_ref[...], preferred_element_type=jnp.float32)
    acc = acc + jnp.dot(n_ref[...], wn_ref[...],
                        preferred_element_type=jnp.float32)
    o_ref[...] = jnp.maximum(acc + b_ref[...], 0.0)


_final_call = pl.pallas_call(
    _final_body,
    grid=(B // BN,),
    in_specs=[
        pl.BlockSpec((BN, D), lambda i: (i, 0)),
        pl.BlockSpec((BN, D), lambda i: (i, 0)),
        pl.BlockSpec((D, D), lambda i: (0, 0)),
        pl.BlockSpec((D, D), lambda i: (0, 0)),
        pl.BlockSpec((1, D), lambda i: (0, 0)),
    ],
    out_specs=pl.BlockSpec((BN, D), lambda i: (i, 0)),
    out_shape=jax.ShapeDtypeStruct((B, D), jnp.float32),
)


def kernel(nodes, history_uv, history_r, feat_table, rating_table,
           agg_W, agg_b, W1, b1):
    nodes = nodes.astype(jnp.int32)
    hu_p = jnp.pad(history_uv.astype(jnp.int32), ((0, 0), (0, LP - L)))
    hr_p = jnp.pad(history_r.astype(jnp.int32), ((0, 0), (0, LP - L)))

    w_uv = agg_W[:D]
    w_aug = jnp.concatenate([agg_W[D:], agg_b[None]], 0)
    rating_aug = jnp.concatenate(
        [rating_table, jnp.ones((R, 1), jnp.float32)], 1)
    rating_aug = jnp.pad(rating_aug, ((0, 8 - R), (0, 0)))

    feat_proj, rating_proj = _prep_call(feat_table, w_uv, rating_aug, w_aug)
    neigh_sum, self_feats = _sc_agg(nodes, hu_p, hr_p,
                                    feat_proj, rating_proj, feat_table)
    wn = W1[:, D:].T * jnp.float32(1.0 / L)
    return _final_call(self_feats, neigh_sum, W1[:, :D].T, wn, b1[None])


# speedup vs baseline: 1.8916x; 1.8916x over previous
import functools

import jax
import jax.numpy as jnp
from jax import lax
from jax.experimental import pallas as pl
from jax.experimental.pallas import tpu as pltpu
from jax.experimental.pallas import tpu_sc as plsc

B, V, L, D, R = 4096, 100000, 50, 64, 5
LP = 56

NC, NS = 2, 16
NW = NC * NS
BPW = B // NW
NCH = D // 16

VB = 2000


def _prep_body(ft_ref, wuv_ref, ra_ref, wa_ref, out_ref):
    fp = jnp.dot(ft_ref[...], wuv_ref[...], preferred_element_type=jnp.float32)
    rp = jnp.dot(ra_ref[...], wa_ref[...], preferred_element_type=jnp.float32)
    comb = fp[:, None, :] + rp[None, :R, :]
    out_ref[...] = comb.reshape(R * VB, D)


_prep_call = pl.pallas_call(
    _prep_body,
    grid=(V // VB,),
    in_specs=[
        pl.BlockSpec((VB, D), lambda i: (i, 0)),
        pl.BlockSpec((D, D), lambda i: (0, 0)),
        pl.BlockSpec((8, D + 1), lambda i: (0, 0)),
        pl.BlockSpec((D + 1, D), lambda i: (0, 0)),
    ],
    out_specs=pl.BlockSpec((R * VB, D), lambda i: (i, 0)),
    out_shape=jax.ShapeDtypeStruct((R * V, D), jnp.float32),
)

HB = 2000


def _fuse_body(hu_ref, hr_ref, hc_ref):
    hc_ref[...] = hu_ref[...] * R + hr_ref[...]


_fuse_call = pl.pallas_call(
    _fuse_body,
    grid=(V // HB,),
    in_specs=[
        pl.BlockSpec((HB, LP), lambda i: (i, 0)),
        pl.BlockSpec((HB, LP), lambda i: (i, 0)),
    ],
    out_specs=pl.BlockSpec((HB, LP), lambda i: (i, 0)),
    out_shape=jax.ShapeDtypeStruct((V, LP), jnp.int32),
)

_sc_mesh = plsc.VectorSubcoreMesh(core_axis_name="c", subcore_axis_name="s")


@functools.partial(
    pl.kernel,
    mesh=_sc_mesh,
    compiler_params=pltpu.CompilerParams(use_tc_tiling_on_sc=False),
    out_type=[
        jax.ShapeDtypeStruct((B, D), jnp.float32),
        jax.ShapeDtypeStruct((B, D), jnp.float32),
    ],
    scratch_types=[
        pltpu.VMEM((BPW,), jnp.int32),
        pltpu.VMEM((BPW, LP), jnp.int32),
        pltpu.VMEM((8, LP, D), jnp.float32),
        pltpu.VMEM((BPW, D), jnp.float32),
        pltpu.VMEM((BPW, D), jnp.float32),
        pltpu.SemaphoreType.DMA,
        pltpu.SemaphoreType.DMA,
    ],
)
def _sc_agg(nodes_hbm, hc_hbm, fpr_hbm, ftab_hbm,
            neigh_hbm, self_hbm,
            idx_v, hc_v, fring, acc_v, sbuf,
            semf, sem2):
    NBUF = 8
    wid = lax.axis_index("s") * NC + lax.axis_index("c")
    base = wid * BPW

    pltpu.sync_copy(nodes_hbm.at[pl.ds(base, BPW)], idx_v)
    self_dma = pltpu.async_copy(ftab_hbm.at[idx_v], sbuf, sem2)
    pltpu.async_copy(hc_hbm.at[idx_v], hc_v, semf).wait()

    zero = jnp.zeros((16,), jnp.float32)

    def _drain(b):
        pltpu.make_async_copy(fpr_hbm.at[hc_v.at[0]], fring.at[b],
                              semf).wait()

    for b in range(NBUF):
        pltpu.async_copy(fpr_hbm.at[hc_v.at[b]], fring.at[b], semf)

    def _group(g, _):
        i0 = g * NBUF
        for b in range(NBUF):
            i = i0 + b
            _drain(b)
            fbuf = fring.at[b]

            def _lstep(l, acc):
                out = []
                for k in range(NCH):
                    s = pl.ds(16 * k, 16)
                    out.append(acc[k] + jnp.maximum(fbuf[l, s], 0.0))
                return tuple(out)

            acc = lax.fori_loop(0, L, _lstep, (zero,) * NCH)
            for k in range(NCH):
                acc_v[i, pl.ds(16 * k, 16)] = acc[k]
            pltpu.async_copy(
                fpr_hbm.at[hc_v.at[jnp.minimum(i + NBUF, BPW - 1)]],
                fring.at[b], semf)
        return 0

    lax.fori_loop(0, BPW // NBUF, _group, 0)

    for b in range(NBUF):
        _drain(b)

    pltpu.sync_copy(acc_v, neigh_hbm.at[pl.ds(base, BPW)])
    self_dma.wait()
    pltpu.sync_copy(sbuf, self_hbm.at[pl.ds(base, BPW)])


BN = 512


def _final_body(s_ref, n_ref, ws_ref, wn_ref, b_ref, o_ref):
    acc = jnp.dot(s_ref[...], ws_ref[...], preferred_element_type=jnp.float32)
    acc = acc + jnp.dot(n_ref[...], wn_ref[...],
                        preferred_element_type=jnp.float32)
    o_ref[...] = jnp.maximum(acc + b_ref[...], 0.0)


_final_call = pl.pallas_call(
    _final_body,
    grid=(B // BN,),
    in_specs=[
        pl.BlockSpec((BN, D), lambda i: (i, 0)),
        pl.BlockSpec((BN, D), lambda i: (i, 0)),
        pl.BlockSpec((D, D), lambda i: (0, 0)),
        pl.BlockSpec((D, D), lambda i: (0, 0)),
        pl.BlockSpec((1, D), lambda i: (0, 0)),
    ],
    out_specs=pl.BlockSpec((BN, D), lambda i: (i, 0)),
    out_shape=jax.ShapeDtypeStruct((B, D), jnp.float32),
)


def kernel(nodes, history_uv, history_r, feat_table, rating_table,
           agg_W, agg_b, W1, b1):
    nodes = nodes.astype(jnp.int32)
    hu_p = jnp.pad(history_uv.astype(jnp.int32), ((0, 0), (0, LP - L)))
    hr_p = jnp.pad(history_r.astype(jnp.int32), ((0, 0), (0, LP - L)))

    w_uv = agg_W[:D]
    w_aug = jnp.concatenate([agg_W[D:], agg_b[None]], 0)
    rating_aug = jnp.concatenate(
        [rating_table, jnp.ones((R, 1), jnp.float32)], 1)
    rating_aug = jnp.pad(rating_aug, ((0, 8 - R), (0, 0)))

    fproj_r = _prep_call(feat_table, w_uv, rating_aug, w_aug)
    hc = _fuse_call(hu_p, hr_p)
    neigh_sum, self_feats = _sc_agg(nodes, hc, fproj_r, feat_table)
    wn = W1[:, D:].T * jnp.float32(1.0 / L)
    return _final_call(self_feats, neigh_sum, W1[:, :D].T, wn, b1[None])


# speedup vs baseline: 1.9094x; 1.0094x over previous
import functools

import jax
import jax.numpy as jnp
from jax import lax
from jax.experimental import pallas as pl
from jax.experimental.pallas import tpu as pltpu
from jax.experimental.pallas import tpu_sc as plsc

B, V, L, D, R = 4096, 100000, 50, 64, 5
LP = 56

NC, NS = 2, 16
NW = NC * NS
BPW = B // NW
NCH = D // 16

VB = 2000


def _prep_body(ft_ref, wuv_ref, ra_ref, wa_ref, out_ref):
    fp = jnp.dot(ft_ref[...], wuv_ref[...], preferred_element_type=jnp.float32)
    rp = jnp.dot(ra_ref[...], wa_ref[...], preferred_element_type=jnp.float32)
    comb = fp[:, None, :] + rp[None, :R, :]
    out_ref[...] = comb.reshape(R * VB, D)


_prep_call = pl.pallas_call(
    _prep_body,
    grid=(V // VB,),
    in_specs=[
        pl.BlockSpec((VB, D), lambda i: (i, 0)),
        pl.BlockSpec((D, D), lambda i: (0, 0)),
        pl.BlockSpec((8, D + 1), lambda i: (0, 0)),
        pl.BlockSpec((D + 1, D), lambda i: (0, 0)),
    ],
    out_specs=pl.BlockSpec((R * VB, D), lambda i: (i, 0)),
    out_shape=jax.ShapeDtypeStruct((R * V, D), jnp.float32),
)

HB = 2000


def _fuse_body(hu_ref, hr_ref, hc_ref):
    hc = hu_ref[...] * R + hr_ref[...]
    hc_ref[...] = jnp.pad(hc, ((0, 0), (0, LP - L)))


_fuse_call = pl.pallas_call(
    _fuse_body,
    grid=(V // HB,),
    in_specs=[
        pl.BlockSpec((HB, L), lambda i: (i, 0)),
        pl.BlockSpec((HB, L), lambda i: (i, 0)),
    ],
    out_specs=pl.BlockSpec((HB, LP), lambda i: (i, 0)),
    out_shape=jax.ShapeDtypeStruct((V, LP), jnp.int32),
)

_sc_mesh = plsc.VectorSubcoreMesh(core_axis_name="c", subcore_axis_name="s")


@functools.partial(
    pl.kernel,
    mesh=_sc_mesh,
    compiler_params=pltpu.CompilerParams(use_tc_tiling_on_sc=False),
    out_type=[
        jax.ShapeDtypeStruct((B, D), jnp.float32),
        jax.ShapeDtypeStruct((B, D), jnp.float32),
    ],
    scratch_types=[
        pltpu.VMEM((BPW,), jnp.int32),
        pltpu.VMEM((BPW, LP), jnp.int32),
        pltpu.VMEM((8, LP, D), jnp.float32),
        pltpu.VMEM((BPW, D), jnp.float32),
        pltpu.VMEM((BPW, D), jnp.float32),
        pltpu.SemaphoreType.DMA,
        pltpu.SemaphoreType.DMA,
    ],
)
def _sc_agg(nodes_hbm, hc_hbm, fpr_hbm, ftab_hbm,
            neigh_hbm, self_hbm,
            idx_v, hc_v, fring, acc_v, sbuf,
            semf, sem2):
    NBUF = 8
    wid = lax.axis_index("s") * NC + lax.axis_index("c")
    base = wid * BPW

    pltpu.sync_copy(nodes_hbm.at[pl.ds(base, BPW)], idx_v)
    self_dma = pltpu.async_copy(ftab_hbm.at[idx_v], sbuf, sem2)
    pltpu.async_copy(hc_hbm.at[idx_v], hc_v, semf).wait()

    zero = jnp.zeros((16,), jnp.float32)

    def _idx(i):
        return hc_v.at[i]

    def _drain(b):
        pltpu.make_async_copy(fpr_hbm.at[_idx(0)], fring.at[b],
                              semf).wait()

    for b in range(NBUF):
        pltpu.async_copy(fpr_hbm.at[_idx(b)], fring.at[b], semf)

    def _group(g, _):
        i0 = g * NBUF
        for b in range(NBUF):
            i = i0 + b
            _drain(b)
            fbuf = fring.at[b]

            def _lstep(l, acc):
                out = []
                for k in range(NCH):
                    s = pl.ds(16 * k, 16)
                    out.append(acc[k] + jnp.maximum(fbuf[l, s], 0.0))
                return tuple(out)

            acc = lax.fori_loop(0, L, _lstep, (zero,) * NCH)
            for k in range(NCH):
                acc_v[i, pl.ds(16 * k, 16)] = acc[k]
            pltpu.async_copy(
                fpr_hbm.at[_idx(jnp.minimum(i + NBUF, BPW - 1))],
                fring.at[b], semf)
        return 0

    lax.fori_loop(0, BPW // NBUF, _group, 0)

    for b in range(NBUF):
        _drain(b)

    pltpu.sync_copy(acc_v, neigh_hbm.at[pl.ds(base, BPW)])
    self_dma.wait()
    pltpu.sync_copy(sbuf, self_hbm.at[pl.ds(base, BPW)])


BN = 512


def _final_body(s_ref, n_ref, ws_ref, wn_ref, b_ref, o_ref):
    acc = jnp.dot(s_ref[...], ws_ref[...], preferred_element_type=jnp.float32)
    acc = acc + jnp.dot(n_ref[...], wn_ref[...],
                        preferred_element_type=jnp.float32)
    o_ref[...] = jnp.maximum(acc + b_ref[...], 0.0)


_final_call = pl.pallas_call(
    _final_body,
    grid=(B // BN,),
    in_specs=[
        pl.BlockSpec((BN, D), lambda i: (i, 0)),
        pl.BlockSpec((BN, D), lambda i: (i, 0)),
        pl.BlockSpec((D, D), lambda i: (0, 0)),
        pl.BlockSpec((D, D), lambda i: (0, 0)),
        pl.BlockSpec((1, D), lambda i: (0, 0)),
    ],
    out_specs=pl.BlockSpec((BN, D), lambda i: (i, 0)),
    out_shape=jax.ShapeDtypeStruct((B, D), jnp.float32),
)


def kernel(nodes, history_uv, history_r, feat_table, rating_table,
           agg_W, agg_b, W1, b1):
    nodes = nodes.astype(jnp.int32)
    hu_p = history_uv.astype(jnp.int32)
    hr_p = history_r.astype(jnp.int32)

    w_uv = agg_W[:D]
    w_aug = jnp.concatenate([agg_W[D:], agg_b[None]], 0)
    rating_aug = jnp.concatenate(
        [rating_table, jnp.ones((R, 1), jnp.float32)], 1)
    rating_aug = jnp.pad(rating_aug, ((0, 8 - R), (0, 0)))

    fproj_r = _prep_call(feat_table, w_uv, rating_aug, w_aug)
    hc = _fuse_call(hu_p, hr_p)
    neigh_sum, self_feats = _sc_agg(nodes, hc, fproj_r, feat_table)
    wn = W1[:, D:].T * jnp.float32(1.0 / L)
    return _final_call(self_feats, neigh_sum, W1[:, :D].T, wn, b1[None])
